# baseline (speedup 1.0000x reference)
import functools

import jax
import jax.numpy as jnp
from jax.experimental import pallas as pl
from jax.experimental.pallas import tpu as pltpu

L = 50
D = 300
H = 4


def _dot(a, b, dims):
    return jax.lax.dot_general(a, b, dimension_numbers=(dims, ((), ())),
                               preferred_element_type=jnp.float32)


def _round(x, eye, wsim_gat, wsim_sage, Wg, al, ar, bg, Ws, bs, whw, bhw):
    e1 = jnp.maximum(x * wsim_gat, 0.0)
    adj1 = _dot(e1, e1, ((1,), (1,)))
    mgat = (adj1 >= 0.1) | eye
    e2 = jnp.maximum(x * wsim_sage, 0.0)
    adj2 = _dot(e2, e2, ((1,), (1,)))
    msage = (adj2 >= 0.1) | eye

    ones_col = jnp.ones((L, 1), jnp.float32)

    chans = []
    for h in range(H):
        z = _dot(x, Wg[h], ((1,), (0,)))
        el = _dot(z, al[h], ((1,), (1,)))
        er = _dot(ar[h], z, ((1,), (1,)))
        e = el + er
        e = jnp.where(e >= 0, e, 0.2 * e)
        e = jnp.where(mgat, e, -1e9)
        emax = jnp.max(e, axis=0, keepdims=True)
        ex = jnp.exp(e - emax)
        alpha = ex / jnp.sum(ex, axis=0, keepdims=True)
        att = _dot(alpha, z, ((0,), (0,)))
        chans.append(att + bg[h])

    m = msage.astype(jnp.float32)
    deg = _dot(m, ones_col, ((0,), (0,)))
    agg = _dot(m, x, ((0,), (0,)))
    hs = (agg + x) / (deg + 1.0)
    chans.append(_dot(hs, Ws, ((1,), (0,))) + bs)

    out = jnp.zeros((L, D), jnp.float32)
    for ch in chans:
        gate = jax.nn.sigmoid(jnp.sum(ch * whw, axis=1, keepdims=True) + bhw)
        out = out + ch * gate
    return x + out


def _main_kernel(x_ref, ws1_ref, ws2_ref, wws_ref, bws_ref,
                 Wg1, al1, ar1, bg1, Ws1, bs1, whw1, bhw1,
                 Wg2, al2, ar2, bg2, Ws2, bs2, whw2, bhw2,
                 out_ref):
    x = x_ref[0]
    iu = jax.lax.broadcasted_iota(jnp.int32, (L, L), 0)
    iv = jax.lax.broadcasted_iota(jnp.int32, (L, L), 1)
    eye = iu == iv
    ws1 = ws1_ref[...]
    ws2 = ws2_ref[...]
    x = _round(x, eye, ws2, ws1, Wg1[...], al1[...], ar1[...], bg1[...],
               Ws1[...], bs1[...], whw1[...], bhw1[...])
    x = _round(x, eye, ws2, ws1, Wg2[...], al2[...], ar2[...], bg2[...],
               Ws2[...], bs2[...], whw2[...], bhw2[...])
    gate = jax.nn.sigmoid(jnp.sum(x * wws_ref[...], axis=1, keepdims=True)
                          + bws_ref[...])
    out_ref[0] = jnp.sum(x * gate, axis=0, keepdims=True)


def _pearson_kernel(p_ref, out_ref):
    p = p_ref[...].reshape(512, D)
    g1 = p[0:256, :]
    g2 = p[256:512, :]
    g1 = g1 - jnp.mean(g1, axis=1, keepdims=True)
    g2 = g2 - jnp.mean(g2, axis=1, keepdims=True)
    num = jnp.sum(g1 * g2, axis=1)
    den = jnp.sqrt(jnp.sum(g1 * g1, axis=1)) * jnp.sqrt(jnp.sum(g2 * g2, axis=1))
    out_ref[...] = num / den * 5.0


def _mg_args(p):
    Wg = p["W_gat"].reshape(D, H, D).transpose(1, 0, 2)
    al = p["attn_l"][:, None, :]
    ar = p["attn_r"][:, None, :]
    bg = p["b_gat"][:, None, :]
    whw = p["W_hw"].reshape(1, D)
    bhw = p["b_hw"].reshape(1, 1)
    return [Wg, al, ar, bg, p["W_sage"], p["b_sage"].reshape(1, D), whw, bhw]


def kernel(sentence_1, sentence_2, emb_table, w_sim1, w_sim2, params):
    BG = 2 * sentence_1.shape[1]
    idx = jnp.concatenate([sentence_1.T.reshape(-1), sentence_2.T.reshape(-1)])
    x = jnp.take(emb_table, idx, axis=0).reshape(BG, L, D)

    full = lambda shape: pl.BlockSpec(shape, lambda i: (0,) * len(shape))
    w_specs = [full((H, D, D)), full((H, 1, D)), full((H, 1, D)), full((H, 1, D)),
               full((D, D)), full((1, D)), full((1, D)), full((1, 1))]
    grid_spec = pl.GridSpec(
        grid=(BG,),
        in_specs=[pl.BlockSpec((1, L, D), lambda i: (i, 0, 0)),
                  full((1, D)), full((1, D)), full((1, D)), full((1, 1))]
                 + w_specs + w_specs,
        out_specs=pl.BlockSpec((1, 1, D), lambda i: (i, 0, 0)),
    )
    pooled = pl.pallas_call(
        _main_kernel,
        grid_spec=grid_spec,
        out_shape=jax.ShapeDtypeStruct((BG, 1, D), jnp.float32),
    )(x, w_sim1.reshape(1, D), w_sim2.reshape(1, D),
      params["W_ws"].reshape(1, D), params["b_ws"].reshape(1, 1),
      *_mg_args(params["mg1"]), *_mg_args(params["mg2"]))

    return pl.pallas_call(
        _pearson_kernel,
        out_shape=jax.ShapeDtypeStruct((BG // 2,), jnp.float32),
    )(pooled)
